# baseline (speedup 1.0000x reference)
import functools

import jax
import jax.numpy as jnp
from jax import lax
from jax.experimental import pallas as pl
from jax.experimental.pallas import tpu as pltpu
from jax.experimental.pallas import tpu_sc as plsc

V = 1000000
D = 64
B = 16384
C = 20
NEG = 3

NC = 2
NS = 16
NW = NC * NS
BPW = B // NW
CHUNK = 64
NSTEPS = BPW // CHUNK
DL = D // 16


def _sc_body(tgt_idx_hbm, ctx_idx_hbm, neg_idx_hbm, ctx_emb_hbm, tgt_emb_hbm,
             pos_hbm, neg_hbm,
             tgt_idx_v, ctx_idx_v, neg_idx_v,
             tgt_rows_v, ctx_rows_v, neg_rows_v,
             pos_out_v, neg_out_v,
             sem_t, sem_c, sem_n):
    cid = lax.axis_index("c")
    sid = lax.axis_index("s")
    wid = sid * NC + cid
    base = wid * BPW

    def step(s, carry):
        ib = base + s * CHUNK
        pltpu.sync_copy(tgt_idx_hbm.at[pl.ds(ib, CHUNK)], tgt_idx_v)
        pltpu.sync_copy(ctx_idx_hbm.at[pl.ds(ib * C, CHUNK * C)], ctx_idx_v)
        pltpu.sync_copy(neg_idx_hbm.at[pl.ds(ib * NEG, CHUNK * NEG)], neg_idx_v)
        cp_t = pltpu.async_copy(tgt_emb_hbm.at[tgt_idx_v], tgt_rows_v, sem_t)
        cp_c = pltpu.async_copy(ctx_emb_hbm.at[ctx_idx_v], ctx_rows_v, sem_c)
        cp_n = pltpu.async_copy(ctx_emb_hbm.at[neg_idx_v], neg_rows_v, sem_n)
        cp_t.wait()
        cp_c.wait()
        cp_n.wait()

        def item(i, carry2):
            pacc = jnp.zeros((16,), jnp.float32)
            nacc = jnp.zeros((16,), jnp.float32)
            for c in range(DL):
                sl = pl.ds(c * 16, 16)
                t = tgt_rows_v[i, sl]
                cs = ctx_rows_v[i * C, sl]
                for j in range(1, C):
                    cs = cs + ctx_rows_v[i * C + j, sl]
                ns = neg_rows_v[i * NEG, sl]
                for j in range(1, NEG):
                    ns = ns + neg_rows_v[i * NEG + j, sl]
                pacc = pacc + cs * t
                nacc = nacc + ns * t
            pos_out_v[i, :] = pacc
            neg_out_v[i, :] = nacc
            return carry2

        lax.fori_loop(0, CHUNK, item, 0, unroll=False)
        pltpu.sync_copy(pos_out_v, pos_hbm.at[pl.ds(ib, CHUNK)])
        pltpu.sync_copy(neg_out_v, neg_hbm.at[pl.ds(ib, CHUNK)])
        return carry

    lax.fori_loop(0, NSTEPS, step, 0, unroll=False)


def _tc_body(pos_ref, neg_ref, out_ref):
    p = jnp.sum(pos_ref[...], axis=1) * (1.0 / C)
    n = -jnp.sum(neg_ref[...], axis=1)

    def logsig(x):
        return jnp.minimum(x, 0.0) - jnp.log1p(jnp.exp(-jnp.abs(x)))

    total = jnp.sum(logsig(p) + logsig(n))
    out_ref[0, 0] = -total * (1.0 / B)


def kernel(targets, contexts, negsamples, context_emb, target_emb):
    tgt_idx = targets.astype(jnp.int32)
    ctx_idx = contexts.astype(jnp.int32).reshape(B * C)
    neg_idx = negsamples.astype(jnp.int32).reshape(B * NEG)

    mesh = plsc.VectorSubcoreMesh(core_axis_name="c", subcore_axis_name="s",
                                  num_cores=NC, num_subcores=NS)
    sc = pl.kernel(
        _sc_body,
        out_type=(jax.ShapeDtypeStruct((B, 16), jnp.float32),
                  jax.ShapeDtypeStruct((B, 16), jnp.float32)),
        mesh=mesh,
        compiler_params=pltpu.CompilerParams(use_tc_tiling_on_sc=False),
        scratch_types=[
            pltpu.VMEM((CHUNK,), jnp.int32),
            pltpu.VMEM((CHUNK * C,), jnp.int32),
            pltpu.VMEM((CHUNK * NEG,), jnp.int32),
            pltpu.VMEM((CHUNK, D), jnp.float32),
            pltpu.VMEM((CHUNK * C, D), jnp.float32),
            pltpu.VMEM((CHUNK * NEG, D), jnp.float32),
            pltpu.VMEM((CHUNK, 16), jnp.float32),
            pltpu.VMEM((CHUNK, 16), jnp.float32),
            pltpu.SemaphoreType.DMA,
            pltpu.SemaphoreType.DMA,
            pltpu.SemaphoreType.DMA,
        ],
    )
    pos_part, neg_part = sc(tgt_idx, ctx_idx, neg_idx, context_emb, target_emb)

    loss = pl.pallas_call(
        _tc_body,
        out_shape=jax.ShapeDtypeStruct((1, 1), jnp.float32),
        in_specs=[pl.BlockSpec(memory_space=pltpu.VMEM),
                  pl.BlockSpec(memory_space=pltpu.VMEM)],
        out_specs=pl.BlockSpec(memory_space=pltpu.SMEM),
    )(pos_part, neg_part)
    return loss
